# baseline (speedup 1.0000x reference)
import functools

import jax
import jax.numpy as jnp
from jax import lax
from jax.experimental import pallas as pl
from jax.experimental.pallas import tpu as pltpu
from jax.experimental.pallas import tpu_sc as plsc

_NEG = -1e30


def _gap_body(x_ref, bat_ref, w1_ref, b1_ref, w2_ref,
              m_ref, s_ref, acc_ref, *, nseg):
    xb = x_ref[...].astype(jnp.bfloat16)
    h = jnp.maximum(
        jnp.dot(xb, w1_ref[...], preferred_element_type=jnp.float32)
        + b1_ref[...], 0.0).astype(jnp.bfloat16)
    g = jnp.dot(h, w2_ref[...],
                preferred_element_type=jnp.float32)

    r = g.shape[0]
    nsplit = 8
    sz = r // nsplit
    parts = [jnp.max(g[j * sz:(j + 1) * sz]) for j in range(nsplit)]
    c = functools.reduce(jnp.maximum, parts)

    gsb = (g - c).astype(jnp.bfloat16)
    bat = bat_ref[0, 0, :]
    seg = lax.broadcasted_iota(jnp.int16, (r, nseg), 1)
    onehot = bat[:, None] == seg
    wb = jnp.exp2(jnp.where(onehot, gsb,
                            jnp.bfloat16(_NEG)))

    dn = (((0,), (0,)), ((), ()))
    acc_blk = lax.dot_general(wb, xb, dn,
                              preferred_element_type=jnp.float32)
    ones = jnp.ones((r, 1), dtype=jnp.bfloat16)
    s_blk = lax.dot_general(wb, ones, dn,
                            preferred_element_type=jnp.float32)

    pres = s_blk > 0.0
    m_ref[...] = jnp.where(pres, c, _NEG).reshape(1, nseg, 1)
    s_ref[...] = s_blk.reshape(1, nseg, 1)
    acc_ref[...] = acc_blk.reshape(1, acc_blk.shape[0], acc_blk.shape[1])


def _partials(x, bat3, w1b, b1v, w2l, *, nblk, r, nseg, nin):
    return pl.pallas_call(
        functools.partial(_gap_body, nseg=nseg),
        grid=(nblk,),
        in_specs=[
            pl.BlockSpec((r, nin), lambda i: (i, 0)),
            pl.BlockSpec((1, 1, r), lambda i: (i, 0, 0)),
            pl.BlockSpec((nin, nin), lambda i: (0, 0)),
            pl.BlockSpec((1, nin), lambda i: (0, 0)),
            pl.BlockSpec((nin, 1), lambda i: (0, 0)),
        ],
        out_specs=[
            pl.BlockSpec((1, nseg, 1), lambda i: (i, 0, 0)),
            pl.BlockSpec((1, nseg, 1), lambda i: (i, 0, 0)),
            pl.BlockSpec((1, nseg, nin), lambda i: (i, 0, 0)),
        ],
        out_shape=[
            jax.ShapeDtypeStruct((nblk, nseg, 1), jnp.float32),
            jax.ShapeDtypeStruct((nblk, nseg, 1), jnp.float32),
            jax.ShapeDtypeStruct((nblk, nseg, nin), jnp.float32),
        ],
        compiler_params=pltpu.CompilerParams(
            dimension_semantics=("arbitrary",)),
    )(x, bat3, w1b, b1v, w2l)


def _make_sc_combine(nblk, nseg, nin):
    info = plsc.get_sparse_core_info()
    nc, ns, nl = info.num_cores, info.num_subcores, info.num_lanes
    nw = nc * ns
    seg_per_w = nseg // nw
    mesh = plsc.VectorSubcoreMesh(core_axis_name="c", subcore_axis_name="s")

    @functools.partial(
        pl.kernel, mesh=mesh,
        out_type=jax.ShapeDtypeStruct((nseg, nin), jnp.float32),
        scratch_types=[
            pltpu.VMEM((nseg * nl,), jnp.float32),
            pltpu.VMEM((nseg * nl,), jnp.float32),
            pltpu.VMEM((nseg * nblk, nin), jnp.float32),
            pltpu.VMEM((nin,), jnp.float32),
        ],
    )
    def combine(m_hbm, s_hbm, acc_hbm, out_hbm, m_v, s_v, acc_v, row_v):
        wid = lax.axis_index("s") * nc + lax.axis_index("c")
        pltpu.sync_copy(m_hbm, m_v)
        pltpu.sync_copy(s_hbm, s_v)
        pltpu.sync_copy(acc_hbm, acc_v)
        for k in range(seg_per_w):
            q = wid * seg_per_w + k
            mvec = m_v[pl.ds(q * nl, nl)]
            big = functools.reduce(jnp.maximum,
                                   [mvec[b] for b in range(nblk)])
            wv = jnp.exp((mvec - big) * jnp.float32(0.6931471805599453))
            svec = s_v[pl.ds(q * nl, nl)]
            stot = functools.reduce(
                lambda a, b_: a + b_,
                [svec[b] * wv[b] for b in range(nblk)])
            stotv = jnp.full((nl,), stot, jnp.float32)
            inv = jnp.where(stotv > 0.0, 1.0 / stotv, 0.0)
            for j in range(nin // nl):
                accj = jnp.zeros((nl,), jnp.float32)
                for b in range(nblk):
                    accj = accj + wv[b] * acc_v[q * nblk + b,
                                                pl.ds(j * nl, nl)]
                row_v[pl.ds(j * nl, nl)] = accj * inv
            pltpu.sync_copy(row_v, out_hbm.at[q])

    return combine


def kernel(x, pos, batch, W1, b1, W2, b2):
    del pos, b2
    n, nin = x.shape
    nseg = 64
    r = 20000
    assert n % r == 0
    nblk = n // r

    bat3 = batch.astype(jnp.int16).reshape(nblk, 1, r)
    b1v = b1.reshape(1, nin)
    w1b = W1.astype(jnp.bfloat16)
    w2l = (W2 * jnp.float32(1.4426950408889634)).astype(jnp.bfloat16)

    m_p, s_p, acc_p = _partials(x, bat3, w1b, b1v, w2l,
                                nblk=nblk, r=r, nseg=nseg, nin=nin)
    nl = 16
    m_t = jnp.pad(m_p.reshape(nblk, nseg).T, ((0, 0), (0, nl - nblk)),
                  constant_values=_NEG).reshape(nseg * nl)
    s_t = jnp.pad(s_p.reshape(nblk, nseg).T, ((0, 0), (0, nl - nblk)),
                  constant_values=0.0).reshape(nseg * nl)
    acc_t = acc_p.transpose(1, 0, 2).reshape(nseg * nblk, nin)
    sc = _make_sc_combine(nblk, nseg, nin)
    return sc(m_t, s_t, acc_t)


# speedup vs baseline: 1.0446x; 1.0446x over previous
import functools

import jax
import jax.numpy as jnp
from jax import lax
from jax.experimental import pallas as pl
from jax.experimental.pallas import tpu as pltpu
from jax.experimental.pallas import tpu_sc as plsc

_NEG = -1e30


def _gap_body(x_ref, bat_ref, w1_ref, b1_ref, w2_ref,
              m_ref, s_ref, acc_ref, *, nseg):
    xb = x_ref[...].astype(jnp.bfloat16)
    h = jnp.maximum(
        jnp.dot(xb, w1_ref[...], preferred_element_type=jnp.float32)
        + b1_ref[...], 0.0).astype(jnp.bfloat16)
    g = jnp.dot(h, w2_ref[...],
                preferred_element_type=jnp.float32)

    r = g.shape[0]
    nsplit = 8
    sz = r // nsplit
    parts = [jnp.max(g[j * sz:(j + 1) * sz]) for j in range(nsplit)]
    c = functools.reduce(jnp.maximum, parts)

    gsb = (g - c).astype(jnp.bfloat16)
    bat = bat_ref[0, 0, :]
    seg = lax.broadcasted_iota(jnp.int16, (r, nseg), 1)
    onehot = bat[:, None] == seg
    wb = jnp.exp2(jnp.where(onehot, gsb,
                            jnp.bfloat16(_NEG)))

    dn = (((0,), (0,)), ((), ()))
    acc_blk = lax.dot_general(wb, xb, dn,
                              preferred_element_type=jnp.float32)
    ones = jnp.ones((r, 1), dtype=jnp.bfloat16)
    s_blk = lax.dot_general(wb, ones, dn,
                            preferred_element_type=jnp.float32)

    pres = s_blk > 0.0
    m_ref[...] = jnp.where(pres, c, _NEG).reshape(1, nseg, 1)
    s_ref[...] = s_blk.reshape(1, nseg, 1)
    acc_ref[...] = acc_blk.reshape(1, acc_blk.shape[0], acc_blk.shape[1])


def _partials(x, bat3, w1b, b1v, w2l, *, nblk, r, nseg, nin):
    return pl.pallas_call(
        functools.partial(_gap_body, nseg=nseg),
        grid=(nblk,),
        in_specs=[
            pl.BlockSpec((r, nin), lambda i: (i, 0)),
            pl.BlockSpec((1, 1, r), lambda i: (i, 0, 0)),
            pl.BlockSpec((nin, nin), lambda i: (0, 0)),
            pl.BlockSpec((1, nin), lambda i: (0, 0)),
            pl.BlockSpec((nin, 1), lambda i: (0, 0)),
        ],
        out_specs=[
            pl.BlockSpec((1, nseg, 1), lambda i: (i, 0, 0)),
            pl.BlockSpec((1, nseg, 1), lambda i: (i, 0, 0)),
            pl.BlockSpec((1, nseg, nin), lambda i: (i, 0, 0)),
        ],
        out_shape=[
            jax.ShapeDtypeStruct((nblk, nseg, 1), jnp.float32),
            jax.ShapeDtypeStruct((nblk, nseg, 1), jnp.float32),
            jax.ShapeDtypeStruct((nblk, nseg, nin), jnp.float32),
        ],
        compiler_params=pltpu.CompilerParams(
            dimension_semantics=("arbitrary",)),
    )(x, bat3, w1b, b1v, w2l)


def _make_sc_combine(nblk, nseg, nin):
    info = plsc.get_sparse_core_info()
    nc, ns, nl = info.num_cores, info.num_subcores, info.num_lanes
    nw = nc * ns
    seg_per_w = nseg // nw
    mesh = plsc.VectorSubcoreMesh(core_axis_name="c", subcore_axis_name="s")

    @functools.partial(
        pl.kernel, mesh=mesh,
        out_type=jax.ShapeDtypeStruct((nseg, nin), jnp.float32),
        scratch_types=[
            pltpu.VMEM((nseg * nl,), jnp.float32),
            pltpu.VMEM((nseg * nl,), jnp.float32),
            pltpu.VMEM((8, nin), jnp.float32),
            pltpu.VMEM((nin,), jnp.float32),
        ],
    )
    def combine(m_hbm, s_hbm, acc_hbm, out_hbm, m_v, s_v, acc_v, row_v):
        wid = lax.axis_index("s") * nc + lax.axis_index("c")
        pltpu.sync_copy(m_hbm, m_v)
        pltpu.sync_copy(s_hbm, s_v)
        for k in range(seg_per_w):
            q = wid * seg_per_w + k
            pltpu.sync_copy(acc_hbm.at[pl.ds(q * 8, 8)], acc_v)
            mvec = m_v[pl.ds(q * nl, nl)]
            big = functools.reduce(jnp.maximum,
                                   [mvec[b] for b in range(nblk)])
            wv = jnp.exp((mvec - big) * jnp.float32(0.6931471805599453))
            svec = s_v[pl.ds(q * nl, nl)]
            stot = functools.reduce(
                lambda a, b_: a + b_,
                [svec[b] * wv[b] for b in range(nblk)])
            stotv = jnp.full((nl,), stot, jnp.float32)
            inv = jnp.where(stotv > 0.0, 1.0 / stotv, 0.0)
            for j in range(nin // nl):
                accj = jnp.zeros((nl,), jnp.float32)
                for b in range(nblk):
                    accj = accj + wv[b] * acc_v[b, pl.ds(j * nl, nl)]
                row_v[pl.ds(j * nl, nl)] = accj * inv
            pltpu.sync_copy(row_v, out_hbm.at[q])

    return combine


def kernel(x, pos, batch, W1, b1, W2, b2):
    del pos, b2
    n, nin = x.shape
    nseg = 64
    r = 20000
    assert n % r == 0
    nblk = n // r

    bat3 = batch.astype(jnp.int16).reshape(nblk, 1, r)
    b1v = b1.reshape(1, nin)
    w1b = W1.astype(jnp.bfloat16)
    w2l = (W2 * jnp.float32(1.4426950408889634)).astype(jnp.bfloat16)

    m_p, s_p, acc_p = _partials(x, bat3, w1b, b1v, w2l,
                                nblk=nblk, r=r, nseg=nseg, nin=nin)
    nl = 16
    m_t = jnp.pad(m_p.reshape(nblk, nseg).T, ((0, 0), (0, nl - nblk)),
                  constant_values=_NEG).reshape(nseg * nl)
    s_t = jnp.pad(s_p.reshape(nblk, nseg).T, ((0, 0), (0, nl - nblk)),
                  constant_values=0.0).reshape(nseg * nl)
    acc_t = jnp.pad(acc_p.transpose(1, 0, 2),
                    ((0, 0), (0, 8 - nblk), (0, 0))).reshape(nseg * 8, nin)
    sc = _make_sc_combine(nblk, nseg, nin)
    return sc(m_t, s_t, acc_t)


# speedup vs baseline: 1.0627x; 1.0173x over previous
import functools

import jax
import jax.numpy as jnp
from jax import lax
from jax.experimental import pallas as pl
from jax.experimental.pallas import tpu as pltpu
from jax.experimental.pallas import tpu_sc as plsc

_NEG = -1e30


def _gap_body(x_ref, bat_ref, w1_ref, b1_ref, w2_ref,
              m_ref, s_ref, acc_ref, *, nseg):
    xb = x_ref[...].astype(jnp.bfloat16)
    h = jnp.maximum(
        jnp.dot(xb, w1_ref[...], preferred_element_type=jnp.float32)
        + b1_ref[...], 0.0).astype(jnp.bfloat16)
    g = jnp.dot(h, w2_ref[...],
                preferred_element_type=jnp.float32)

    r = g.shape[0]
    nsplit = 8
    sz = r // nsplit
    parts = [jnp.max(g[j * sz:(j + 1) * sz]) for j in range(nsplit)]
    c = functools.reduce(jnp.maximum, parts)

    gsb = (g - c).astype(jnp.bfloat16)
    bat = bat_ref[0, 0, :]
    seg = lax.broadcasted_iota(jnp.int16, (r, nseg), 1)
    onehot = bat[:, None] == seg
    wb = jnp.exp2(jnp.where(onehot, gsb,
                            jnp.bfloat16(_NEG)))

    dn = (((0,), (0,)), ((), ()))
    acc_blk = lax.dot_general(wb, xb, dn,
                              preferred_element_type=jnp.float32)
    ones = jnp.ones((r, 1), dtype=jnp.bfloat16)
    s_blk = lax.dot_general(wb, ones, dn,
                            preferred_element_type=jnp.float32)

    pres = s_blk > 0.0
    m_ref[...] = jnp.where(pres, c, _NEG).reshape(1, nseg, 1)
    s_ref[...] = s_blk.reshape(1, nseg, 1)
    acc_ref[...] = acc_blk.reshape(1, acc_blk.shape[0], acc_blk.shape[1])


def _partials(x, bat3, w1b, b1v, w2l, *, nblk, r, nseg, nin):
    return pl.pallas_call(
        functools.partial(_gap_body, nseg=nseg),
        grid=(nblk,),
        in_specs=[
            pl.BlockSpec((r, nin), lambda i: (i, 0)),
            pl.BlockSpec((1, 1, r), lambda i: (i, 0, 0)),
            pl.BlockSpec((nin, nin), lambda i: (0, 0)),
            pl.BlockSpec((1, nin), lambda i: (0, 0)),
            pl.BlockSpec((nin, 1), lambda i: (0, 0)),
        ],
        out_specs=[
            pl.BlockSpec((1, nseg, 1), lambda i: (i, 0, 0)),
            pl.BlockSpec((1, nseg, 1), lambda i: (i, 0, 0)),
            pl.BlockSpec((1, nseg, nin), lambda i: (i, 0, 0)),
        ],
        out_shape=[
            jax.ShapeDtypeStruct((nblk, nseg, 1), jnp.float32),
            jax.ShapeDtypeStruct((nblk, nseg, 1), jnp.float32),
            jax.ShapeDtypeStruct((nblk, nseg, nin), jnp.float32),
        ],
        compiler_params=pltpu.CompilerParams(
            dimension_semantics=("arbitrary",)),
    )(x, bat3, w1b, b1v, w2l)


def _make_sc_combine(nblk, nseg, nin):
    info = plsc.get_sparse_core_info()
    nc, ns, nl = info.num_cores, info.num_subcores, info.num_lanes
    nw = nc * ns
    seg_per_w = nseg // nw
    mesh = plsc.VectorSubcoreMesh(core_axis_name="c", subcore_axis_name="s")

    @functools.partial(
        pl.kernel, mesh=mesh,
        out_type=jax.ShapeDtypeStruct((nseg, nin), jnp.float32),
        scratch_types=[
            pltpu.VMEM((2 * nseg * nl,), jnp.float32),
            pltpu.VMEM((8, nin), jnp.float32),
            pltpu.VMEM((nin,), jnp.float32),
        ],
    )
    def combine(ms_hbm, acc_hbm, out_hbm, ms_v, acc_v, row_v):
        wid = lax.axis_index("s") * nc + lax.axis_index("c")
        pltpu.sync_copy(ms_hbm, ms_v)
        for k in range(seg_per_w):
            q = wid * seg_per_w + k
            pltpu.sync_copy(acc_hbm.at[pl.ds(q * 8, 8)], acc_v)
            mvec = ms_v[pl.ds(q * nl, nl)]
            big = functools.reduce(jnp.maximum,
                                   [mvec[b] for b in range(nblk)])
            wv = jnp.exp((mvec - big) * jnp.float32(0.6931471805599453))
            svec = ms_v[pl.ds(nseg * nl + q * nl, nl)]
            stot = functools.reduce(
                lambda a, b_: a + b_,
                [svec[b] * wv[b] for b in range(nblk)])
            stotv = jnp.full((nl,), stot, jnp.float32)
            inv = jnp.where(stotv > 0.0, 1.0 / stotv, 0.0)
            for j in range(nin // nl):
                accj = jnp.zeros((nl,), jnp.float32)
                for b in range(nblk):
                    accj = accj + wv[b] * acc_v[b, pl.ds(j * nl, nl)]
                row_v[pl.ds(j * nl, nl)] = accj * inv
            pltpu.sync_copy(row_v, out_hbm.at[q])

    return combine


def kernel(x, pos, batch, W1, b1, W2, b2):
    del pos, b2
    n, nin = x.shape
    nseg = 64
    r = 20000
    assert n % r == 0
    nblk = n // r

    bat3 = batch.astype(jnp.int16).reshape(nblk, 1, r)
    b1v = b1.reshape(1, nin)
    w1b = W1.astype(jnp.bfloat16)
    w2l = (W2 * jnp.float32(1.4426950408889634)).astype(jnp.bfloat16)

    m_p, s_p, acc_p = _partials(x, bat3, w1b, b1v, w2l,
                                nblk=nblk, r=r, nseg=nseg, nin=nin)
    nl = 16
    ms2 = jnp.stack([m_p.reshape(nblk, nseg).T, s_p.reshape(nblk, nseg).T])
    padv = jnp.full((2, 1, 1), jnp.float32(_NEG)).at[1].set(0.0)
    ms_t = jnp.concatenate(
        [ms2, jnp.broadcast_to(padv, (2, nseg, nl - nblk))],
        axis=2).reshape(2 * nseg * nl)
    acc_t = jnp.pad(acc_p.transpose(1, 0, 2),
                    ((0, 0), (0, 8 - nblk), (0, 0))).reshape(nseg * 8, nin)
    sc = _make_sc_combine(nblk, nseg, nin)
    return sc(ms_t, acc_t)
